# baseline (speedup 1.0000x reference)
import jax
import jax.numpy as jnp
from jax.experimental import pallas as pl
from jax.experimental.pallas import tpu as pltpu
from jax.experimental.pallas import tpu_sc as plsc

H = 128
W = 128
CF = 16
K = 8
N = H * W
BQ = 128
CW = 128
NT = N // CW


def _conv(x, w, b):
    y = jax.lax.conv_general_dilated(
        x, w, (1, 1), 'SAME', dimension_numbers=('NCHW', 'OIHW', 'NCHW'))
    return y + b[None, :, None, None]


def _topk_body(q_ref, kt_ref, wgt_ref, idx_ref, cost_scr):
    q = q_ref[...]
    q2 = jnp.sum(q * q, axis=1, keepdims=True)

    def mm_step(t, carry):
        kt = kt_ref[t]
        k2 = jnp.sum(kt * kt, axis=0, keepdims=True)
        c = q2 - 2.0 * jnp.dot(q, kt, preferred_element_type=jnp.float32) + k2
        cost_scr[t] = c
        return carry

    jax.lax.fori_loop(0, NT, mm_step, 0, unroll=4)

    inf = jnp.float32(jnp.inf)
    big = jnp.int32(2 ** 30)
    lane = jax.lax.broadcasted_iota(jnp.int32, (BQ, CW), 1)

    ms = []
    idxs = []
    m_prev = jnp.full((BQ, 1), -inf, dtype=jnp.float32)
    i_prev = jnp.full((BQ, 1), -1, dtype=jnp.int32)
    for k in range(K):
        def ext_step(t, carry):
            acc_v, acc_i = carry
            c = cost_scr[t]
            gcol = lane + t * CW
            valid = (c > m_prev) | ((c == m_prev) & (gcol > i_prev))
            ceff = jnp.where(valid, c, inf)
            take = ceff < acc_v
            acc_i = jnp.where(take, gcol, acc_i)
            acc_v = jnp.where(take, ceff, acc_v)
            return acc_v, acc_i

        acc_v0 = jnp.full((BQ, CW), inf, dtype=jnp.float32)
        acc_i0 = jnp.full((BQ, CW), big, dtype=jnp.int32)
        acc_v, acc_i = jax.lax.fori_loop(0, NT, ext_step, (acc_v0, acc_i0),
                                         unroll=4)
        m = jnp.min(acc_v, axis=1, keepdims=True)
        i = jnp.min(jnp.where(acc_v == m, acc_i, big), axis=1,
                    keepdims=True)
        ms.append(m)
        idxs.append(i)
        m_prev, i_prev = m, i

    costs = jnp.concatenate(ms, axis=1)
    ids = jnp.concatenate(idxs, axis=1)
    e = jnp.exp(costs[:, 0:1] - costs)
    wgt_ref[...] = e / jnp.sum(e, axis=1, keepdims=True)
    idx_ref[...] = ids


def _topk(qf, kt3):
    grid = (N // BQ,)
    return pl.pallas_call(
        _topk_body,
        grid=grid,
        in_specs=[
            pl.BlockSpec((BQ, CF), lambda i: (i, 0)),
            pl.BlockSpec((NT, CF, CW), lambda i: (0, 0, 0)),
        ],
        out_specs=[
            pl.BlockSpec((BQ, K), lambda i: (i, 0)),
            pl.BlockSpec((BQ, K), lambda i: (i, 0)),
        ],
        out_shape=[
            jax.ShapeDtypeStruct((N, K), jnp.float32),
            jax.ShapeDtypeStruct((N, K), jnp.int32),
        ],
        scratch_shapes=[pltpu.VMEM((NT, BQ, CW), jnp.float32)],
    )(qf, kt3)


def _sc_gather(vT, idx):
    NI = N * K
    GW = 128
    VD = 128
    vpad = jnp.pad(vT, ((0, 0), (0, VD - CF)))
    mesh = plsc.VectorSubcoreMesh(core_axis_name="core",
                                  subcore_axis_name="subcore")

    @pl.kernel(out_type=jax.ShapeDtypeStruct((NI, VD), jnp.float32),
               mesh=mesh)
    def gk(v_hbm, i_hbm, o_hbm):
        def body(i_vmem, o_vmem):
            pltpu.sync_copy(v_hbm.at[i_vmem.at[0]], o_vmem)

        pltpu.emit_pipeline(
            body,
            grid=(NI // GW,),
            in_specs=[pl.BlockSpec((1, GW), index_map=lambda i: (0, i))],
            out_specs=[pl.BlockSpec((GW, VD), index_map=lambda i: (i, 0))],
            core_axis_name='subcore',
            dimension_semantics=(pltpu.PARALLEL,),
        )(i_hbm, o_hbm)

    return gk(vpad, idx.reshape(1, NI))[:, :CF]


def kernel(a, b, Wq, bq, Wk, bk, Wv, bv, Wf, bf):
    q = jax.nn.relu(_conv(a, Wq, bq))[0]
    k = jax.nn.relu(_conv(b, Wk, bk))[0]
    v = jax.nn.relu(_conv(b, Wv, bv))[0]

    qf = q.reshape(CF, N).T
    kt3 = k.reshape(CF, NT, CW).transpose(1, 0, 2)
    vT = v.reshape(CF, N).T

    wgt, idx = _topk(qf, kt3)

    g = _sc_gather(vT, idx)
    att = jnp.sum(wgt[:, :, None] * g.reshape(N, K, CF), axis=1)
    att = att.T.reshape(1, CF, H, W)

    out = jax.nn.sigmoid(_conv(jnp.concatenate([a, att], axis=1), Wf, bf))
    return out


# speedup vs baseline: 1.0276x; 1.0276x over previous
import jax
import jax.numpy as jnp
from jax.experimental import pallas as pl
from jax.experimental.pallas import tpu as pltpu
from jax.experimental.pallas import tpu_sc as plsc

H = 128
W = 128
CF = 16
K = 8
N = H * W
BQ = 128
CW = 128
NT = N // CW


def _conv(x, w, b):
    y = jax.lax.conv_general_dilated(
        x, w, (1, 1), 'SAME', dimension_numbers=('NCHW', 'OIHW', 'NCHW'))
    return y + b[None, :, None, None]


def _topk_body(q_ref, kt_ref, wgt_ref, idx_ref, cost_scr):
    q = q_ref[...]
    q2 = jnp.sum(q * q, axis=1, keepdims=True)

    def mm_step(t, carry):
        kt = kt_ref[t]
        k2 = jnp.sum(kt * kt, axis=0, keepdims=True)
        c = q2 - 2.0 * jnp.dot(q, kt, preferred_element_type=jnp.float32) + k2
        cost_scr[t] = c
        return carry

    jax.lax.fori_loop(0, NT, mm_step, 0, unroll=4)

    inf = jnp.float32(jnp.inf)
    big = jnp.int32(2 ** 30)
    lane = jax.lax.broadcasted_iota(jnp.int32, (BQ, CW), 1)

    ms = []
    idxs = []
    m_prev = jnp.full((BQ, 1), -inf, dtype=jnp.float32)
    i_prev = jnp.full((BQ, 1), -1, dtype=jnp.int32)
    for k in range(K):
        def ext_step(t, carry):
            acc_v, acc_i = carry
            c = cost_scr[t]
            gcol = lane + t * CW
            valid = (c > m_prev) | ((c == m_prev) & (gcol > i_prev))
            ceff = jnp.where(valid, c, inf)
            take = ceff < acc_v
            acc_i = jnp.where(take, gcol, acc_i)
            acc_v = jnp.where(take, ceff, acc_v)
            return acc_v, acc_i

        acc_v0 = jnp.full((BQ, CW), inf, dtype=jnp.float32)
        acc_i0 = jnp.full((BQ, CW), big, dtype=jnp.int32)
        acc_v, acc_i = jax.lax.fori_loop(0, NT, ext_step, (acc_v0, acc_i0),
                                         unroll=4)
        m = jnp.min(acc_v, axis=1, keepdims=True)
        i = jnp.min(jnp.where(acc_v == m, acc_i, big), axis=1,
                    keepdims=True)
        ms.append(m)
        idxs.append(i)
        m_prev, i_prev = m, i

    costs = jnp.concatenate(ms, axis=1)
    ids = jnp.concatenate(idxs, axis=1)
    e = jnp.exp(costs[:, 0:1] - costs)
    wgt_ref[...] = e / jnp.sum(e, axis=1, keepdims=True)
    idx_ref[...] = ids


def _topk(qf, kt3):
    grid = (N // BQ,)
    return pl.pallas_call(
        _topk_body,
        grid=grid,
        in_specs=[
            pl.BlockSpec((BQ, CF), lambda i: (i, 0)),
            pl.BlockSpec((NT, CF, CW), lambda i: (0, 0, 0)),
        ],
        out_specs=[
            pl.BlockSpec((BQ, K), lambda i: (i, 0)),
            pl.BlockSpec((BQ, K), lambda i: (i, 0)),
        ],
        out_shape=[
            jax.ShapeDtypeStruct((N, K), jnp.float32),
            jax.ShapeDtypeStruct((N, K), jnp.int32),
        ],
        scratch_shapes=[pltpu.VMEM((NT, BQ, CW), jnp.float32)],
    )(qf, kt3)


def _sc_att(vT, idx, wgt):
    NI = N * K
    QW = 16
    VD = 128
    vpad = jnp.pad(vT, ((0, 0), (0, VD - CF)))
    wB = jnp.broadcast_to(wgt.reshape(N, K, 1), (N, K, CF)).reshape(N, K * CF)
    mesh = plsc.VectorSubcoreMesh(core_axis_name="core",
                                  subcore_axis_name="subcore")

    @pl.kernel(out_type=jax.ShapeDtypeStruct((N // 8, VD), jnp.float32),
               mesh=mesh,
               scratch_types=[pltpu.VMEM((QW * K, VD), jnp.float32)])
    def gk(v_hbm, i_hbm, w_hbm, o_hbm, g_scr):
        def body(i_vmem, w_vmem, o_vmem):
            pltpu.sync_copy(v_hbm.at[i_vmem.at[0]], g_scr)

            @pl.loop(0, QW // 8)
            def _(q0):
                for qq in range(8):
                    q = q0 * 8 + qq
                    acc = w_vmem[q, 0:CF] * g_scr[q * K, 0:CF]
                    for k in range(1, K):
                        acc = acc + (w_vmem[q, k * CF:(k + 1) * CF]
                                     * g_scr[q * K + k, 0:CF])
                    o_vmem[q0, qq * CF:(qq + 1) * CF] = acc

        pltpu.emit_pipeline(
            body,
            grid=(NI // (QW * K),),
            in_specs=[pl.BlockSpec((1, QW * K), index_map=lambda i: (0, i)),
                      pl.BlockSpec((QW, VD), index_map=lambda i: (i, 0))],
            out_specs=[pl.BlockSpec((QW // 8, VD), index_map=lambda i: (i, 0))],
            core_axis_name='subcore',
            dimension_semantics=(pltpu.PARALLEL,),
        )(i_hbm, w_hbm, o_hbm)

    out = gk(vpad, idx.reshape(1, NI), wB)
    return out.reshape(N, CF)


def kernel(a, b, Wq, bq, Wk, bk, Wv, bv, Wf, bf):
    q = jax.nn.relu(_conv(a, Wq, bq))[0]
    k = jax.nn.relu(_conv(b, Wk, bk))[0]
    v = jax.nn.relu(_conv(b, Wv, bv))[0]

    qf = q.reshape(CF, N).T
    kt3 = k.reshape(CF, NT, CW).transpose(1, 0, 2)
    vT = v.reshape(CF, N).T

    wgt, idx = _topk(qf, kt3)

    att = _sc_att(vT, idx, wgt)
    att = att.T.reshape(1, CF, H, W)

    out = jax.nn.sigmoid(_conv(jnp.concatenate([a, att], axis=1), Wf, bf))
    return out
